# Initial kernel scaffold; baseline (speedup 1.0000x reference)
import jax
import jax.numpy as jnp
from jax.experimental import pallas as pl


def kernel(x, edge_index, edge_attr):
    raise NotImplementedError("write your pallas kernel here")


# speedup vs baseline: 2.9394x; 2.9394x over previous
import jax
import jax.numpy as jnp
from jax import lax
from jax.experimental import pallas as pl
from jax.experimental.pallas import tpu as pltpu
from jax.experimental.pallas import tpu_sc as plsc

_NC = 2
_NS = 16
_NW = _NC * _NS
_IDX = 128
_GRP = 8


def _make_scatter(n_nodes, chunks):
    mesh = plsc.VectorSubcoreMesh(
        core_axis_name="c", subcore_axis_name="s",
        num_cores=_NC, num_subcores=_NS,
    )

    def body(a_ref, src_ref, dst_ref, w_ref, sv, dv, kv, wv, sem):
        wid = lax.axis_index("s") * _NC + lax.axis_index("c")
        pltpu.sync_copy(src_ref.at[wid], sv)
        pltpu.sync_copy(dst_ref.at[wid], dv)
        pltpu.sync_copy(w_ref.at[wid], wv)

        @pl.loop(0, chunks)
        def _keys(j):
            for c in range(_IDX // 16):
                s16 = sv[j, pl.ds(c * 16, 16)]
                d16 = dv[j, pl.ds(c * 16, 16)]
                kv[j, pl.ds(c * 16, 16)] = s16 * n_nodes + d16

        @pl.loop(0, chunks // _GRP)
        def _scatter(g):
            copies = []
            for u in range(_GRP):
                j = g * _GRP + u
                copies.append(
                    pltpu.async_copy(wv.at[j], a_ref.at[kv.at[j]], sem))
            for cp in copies:
                cp.wait()

    return pl.kernel(
        body,
        out_type=(),
        mesh=mesh,
        scratch_types=[
            pltpu.VMEM((chunks, _IDX), jnp.int32),
            pltpu.VMEM((chunks, _IDX), jnp.int32),
            pltpu.VMEM((chunks, _IDX), jnp.int32),
            pltpu.VMEM((chunks, _IDX), jnp.float32),
            pltpu.SemaphoreType.DMA,
        ],
    )


def _rowsum(a):
    n = a.shape[0]
    blk = 128

    def body(a_ref, o_ref):
        o_ref[...] = jnp.sum(a_ref[...], axis=1, keepdims=True)

    return pl.pallas_call(
        body,
        grid=(n // blk,),
        in_specs=[pl.BlockSpec((blk, n), lambda i: (i, 0))],
        out_specs=pl.BlockSpec((blk, 1), lambda i: (i, 0)),
        out_shape=jax.ShapeDtypeStruct((n, 1), jnp.float32),
    )(a)


def _finish(rs, x):
    n, d = x.shape
    blk = 128

    def body(rs_full_ref, rs_ref, x_ref, o_ref):
        cl_full = 1.0 / rs_full_ref[...]
        mn = jnp.min(cl_full)
        mx = jnp.max(cl_full)
        cl = 1.0 / rs_ref[...]
        emb = (cl - mn) / (mx - mn + 1e-08)
        o_ref[...] = x_ref[...] + emb

    return pl.pallas_call(
        body,
        grid=(n // blk,),
        in_specs=[
            pl.BlockSpec((n, 1), lambda i: (0, 0)),
            pl.BlockSpec((blk, 1), lambda i: (i, 0)),
            pl.BlockSpec((blk, d), lambda i: (i, 0)),
        ],
        out_specs=pl.BlockSpec((blk, d), lambda i: (i, 0)),
        out_shape=jax.ShapeDtypeStruct((n, d), jnp.float32),
    )(rs, rs, x)


def kernel(x, edge_index, edge_attr):
    n, _ = x.shape
    e = edge_index.shape[1]
    per_w = e // _NW
    chunks = per_w // _IDX

    w = edge_attr[:, 0]
    e0 = edge_index[0].reshape(_NW, chunks, _IDX)
    e1 = edge_index[1].reshape(_NW, chunks, _IDX)
    wr = w.reshape(_NW, chunks, _IDX)

    scatter = _make_scatter(n, chunks)
    a_ref = jax.new_ref(jnp.zeros((n * n,), jnp.float32))
    scatter(a_ref, e0, e1, wr)
    scatter(a_ref, e1, e0, wr)
    a = a_ref[...].reshape(n, n)

    rs = _rowsum(a)
    return _finish(rs, x)
